# baseline (speedup 1.0000x reference)
import math

import jax
import jax.numpy as jnp
from jax.experimental import pallas as pl


_D_MODEL = 1024
_HALF = _D_MODEL // 2
_SCALE = -math.log(10000.0) / (_HALF - 1)
_T = 32


def _pe_add_body(x_ref, o_ref, *, blk, seq_len):
    i = pl.program_id(0)
    base = jax.lax.rem(i * blk, seq_len)
    q_grp = blk // _T
    j = jax.lax.broadcasted_iota(jnp.int32, (1, 1, _HALF), 2).astype(jnp.float32)
    div = jnp.exp(j * _SCALE)
    alpha = (
        jax.lax.broadcasted_iota(jnp.int32, (q_grp, 1, _HALF), 0) * _T + base
    ).astype(jnp.float32) * div
    beta = (
        jax.lax.broadcasted_iota(jnp.int32, (1, _T, _HALF), 1)
    ).astype(jnp.float32) * div
    sa, ca = jnp.sin(alpha), jnp.cos(alpha)
    sb, cb = jnp.sin(beta), jnp.cos(beta)
    pe_sin = (sa * cb + ca * sb).reshape(blk, _HALF)
    pe_cos = (ca * cb - sa * sb).reshape(blk, _HALF)
    o_ref[:, :_HALF] = x_ref[:, :_HALF] + pe_sin
    o_ref[:, _HALF:] = x_ref[:, _HALF:] + pe_cos


def kernel(inputs):
    batch, seq_len, d_model = inputs.shape
    assert d_model == _D_MODEL
    blk = 2048
    rows = batch * seq_len
    x = inputs.reshape(rows, d_model)
    body = lambda x_ref, o_ref: _pe_add_body(x_ref, o_ref, blk=blk, seq_len=seq_len)
    out = pl.pallas_call(
        body,
        grid=(rows // blk,),
        in_specs=[pl.BlockSpec((blk, d_model), lambda i: (i, 0))],
        out_specs=pl.BlockSpec((blk, d_model), lambda i: (i, 0)),
        out_shape=jax.ShapeDtypeStruct((rows, d_model), inputs.dtype),
    )(x)
    return out.reshape(batch, seq_len, d_model)


# speedup vs baseline: 1.0368x; 1.0368x over previous
import math

import jax
import jax.numpy as jnp
from jax.experimental import pallas as pl
from jax.experimental.pallas import tpu as pltpu


_D_MODEL = 1024
_HALF = _D_MODEL // 2
_SCALE = -math.log(10000.0) / (_HALF - 1)
_T = 32
_BLK = 2048


def _pe_add_body(x_ref, o_ref, psin_ref, pcos_ref):
    h = pl.program_id(0)
    b = pl.program_id(1)

    @pl.when(b == 0)
    def _compute_pe():
        q_grp = _BLK // _T
        j = jax.lax.broadcasted_iota(jnp.int32, (1, 1, _HALF), 2).astype(
            jnp.float32
        )
        div = jnp.exp(j * _SCALE)
        alpha = (
            jax.lax.broadcasted_iota(jnp.int32, (q_grp, 1, _HALF), 0) * _T
            + h * _BLK
        ).astype(jnp.float32) * div
        beta = (
            jax.lax.broadcasted_iota(jnp.int32, (1, _T, _HALF), 1)
        ).astype(jnp.float32) * div
        sa, ca = jnp.sin(alpha), jnp.cos(alpha)
        sb, cb = jnp.sin(beta), jnp.cos(beta)
        psin_ref[...] = (sa * cb + ca * sb).reshape(_BLK, _HALF)
        pcos_ref[...] = (ca * cb - sa * sb).reshape(_BLK, _HALF)

    o_ref[0, :, :_HALF] = x_ref[0, :, :_HALF] + psin_ref[...]
    o_ref[0, :, _HALF:] = x_ref[0, :, _HALF:] + pcos_ref[...]


def kernel(inputs):
    batch, seq_len, d_model = inputs.shape
    assert d_model == _D_MODEL and seq_len % _BLK == 0
    return pl.pallas_call(
        _pe_add_body,
        grid=(seq_len // _BLK, batch),
        in_specs=[
            pl.BlockSpec((1, _BLK, d_model), lambda h, b: (b, h, 0)),
        ],
        out_specs=pl.BlockSpec((1, _BLK, d_model), lambda h, b: (b, h, 0)),
        out_shape=jax.ShapeDtypeStruct(inputs.shape, inputs.dtype),
        scratch_shapes=[
            pltpu.VMEM((_BLK, _HALF), jnp.float32),
            pltpu.VMEM((_BLK, _HALF), jnp.float32),
        ],
    )(inputs)
